# baseline (device time: 24591 ns/iter reference)
import jax
import jax.numpy as jnp
from jax import lax
from jax.experimental import pallas as pl
from jax.experimental.pallas import tpu as pltpu

N_DEV = 8
N_EXP = 16
E_LOC = 2
T_LOC = 256
D_IN = 128
D_OUT = 256
CAP = 102.0
LANES = 128


def kernel(x, router_W, route_idx, expert_W):
    def body(x_ref, rW_ref, idx_ref, ew_ref, out_ref,
             ew_all, cnt_all,
             ew_send_sems, ew_recv_sems, cnt_send_sems, cnt_recv_sems):
        p = lax.axis_index("i")

        idx = idx_ref[:, :]
        iota_e = lax.broadcasted_iota(jnp.int32, (T_LOC, LANES), 1)
        oh = (idx == iota_e).astype(jnp.float32)
        counts = jnp.sum(oh, axis=0, keepdims=True)
        cnt_all[pl.ds(p, 1), :] = counts
        ew_all[pl.ds(p * E_LOC, E_LOC), :, :] = ew_ref[:, :, :]

        barrier_sem = pltpu.get_barrier_semaphore()
        for k in range(1, N_DEV):
            dst = lax.rem(p + k, N_DEV)
            pl.semaphore_signal(
                barrier_sem, inc=1,
                device_id=(dst,), device_id_type=pl.DeviceIdType.MESH,
            )
        pl.semaphore_wait(barrier_sem, N_DEV - 1)

        sends = []
        for k in range(1, N_DEV):
            dst = lax.rem(p + k, N_DEV)
            cnt_rdma = pltpu.make_async_remote_copy(
                src_ref=cnt_all.at[pl.ds(p, 1)],
                dst_ref=cnt_all.at[pl.ds(p, 1)],
                send_sem=cnt_send_sems.at[k - 1],
                recv_sem=cnt_recv_sems.at[k - 1],
                device_id=(dst,), device_id_type=pl.DeviceIdType.MESH,
            )
            cnt_rdma.start()
            ew_rdma = pltpu.make_async_remote_copy(
                src_ref=ew_all.at[pl.ds(p * E_LOC, E_LOC)],
                dst_ref=ew_all.at[pl.ds(p * E_LOC, E_LOC)],
                send_sem=ew_send_sems.at[k - 1],
                recv_sem=ew_recv_sems.at[k - 1],
                device_id=(dst,), device_id_type=pl.DeviceIdType.MESH,
            )
            ew_rdma.start()
            sends.append((cnt_rdma, ew_rdma))

        x_ = x_ref[:, :]
        tri = (
            lax.broadcasted_iota(jnp.int32, (T_LOC, T_LOC), 0)
            > lax.broadcasted_iota(jnp.int32, (T_LOC, T_LOC), 1)
        ).astype(jnp.float32)
        lcb = jnp.dot(tri, oh, preferred_element_type=jnp.float32)

        for k in range(1, N_DEV):
            src_dev = lax.rem(p - k + N_DEV, N_DEV)
            cnt_recv = pltpu.make_async_remote_copy(
                src_ref=cnt_all.at[pl.ds(src_dev, 1)],
                dst_ref=cnt_all.at[pl.ds(src_dev, 1)],
                send_sem=cnt_send_sems.at[k - 1],
                recv_sem=cnt_recv_sems.at[k - 1],
                device_id=(src_dev,), device_id_type=pl.DeviceIdType.MESH,
            )
            cnt_recv.wait_recv()
            ew_recv = pltpu.make_async_remote_copy(
                src_ref=ew_all.at[pl.ds(src_dev * E_LOC, E_LOC)],
                dst_ref=ew_all.at[pl.ds(src_dev * E_LOC, E_LOC)],
                send_sem=ew_send_sems.at[k - 1],
                recv_sem=ew_recv_sems.at[k - 1],
                device_id=(src_dev,), device_id_type=pl.DeviceIdType.MESH,
            )
            ew_recv.wait_recv()

        dev_before = (
            lax.broadcasted_iota(jnp.int32, (N_DEV, 1), 0) < p
        ).astype(jnp.float32)
        offsets = jnp.sum(cnt_all[:, :] * dev_before, axis=0, keepdims=True)
        keep = oh * (lcb + offsets < CAP).astype(jnp.float32)

        acc = jnp.zeros((T_LOC, D_OUT), jnp.float32)
        for e in range(N_EXP):
            acc = acc + jnp.dot(
                x_ * keep[:, e:e + 1], ew_all[e, :, :],
                preferred_element_type=jnp.float32,
            )
        out_ref[:, :] = acc

        for cnt_rdma, ew_rdma in sends:
            cnt_rdma.wait_send()
            ew_rdma.wait_send()

    return pl.pallas_call(
        body,
        out_shape=jax.ShapeDtypeStruct((T_LOC, D_OUT), jnp.float32),
        in_specs=[
            pl.BlockSpec(memory_space=pltpu.VMEM),
            pl.BlockSpec(memory_space=pltpu.VMEM),
            pl.BlockSpec(memory_space=pltpu.VMEM),
            pl.BlockSpec(memory_space=pltpu.VMEM),
        ],
        out_specs=pl.BlockSpec(memory_space=pltpu.VMEM),
        scratch_shapes=[
            pltpu.VMEM((N_EXP, D_IN, D_OUT), jnp.float32),
            pltpu.VMEM((N_DEV, LANES), jnp.float32),
            pltpu.SemaphoreType.DMA((N_DEV - 1,)),
            pltpu.SemaphoreType.DMA((N_DEV - 1,)),
            pltpu.SemaphoreType.DMA((N_DEV - 1,)),
            pltpu.SemaphoreType.DMA((N_DEV - 1,)),
        ],
        compiler_params=pltpu.CompilerParams(collective_id=0),
    )(x, router_W, route_idx, expert_W)


# device time: 23753 ns/iter; 1.0353x vs baseline; 1.0353x over previous
import jax
import jax.numpy as jnp
from jax import lax
from jax.experimental import pallas as pl
from jax.experimental.pallas import tpu as pltpu

N_DEV = 8
N_EXP = 16
E_LOC = 2
T_LOC = 256
D_IN = 128
D_OUT = 256
CAP = 102.0
LANES = 128


def kernel(x, router_W, route_idx, expert_W):
    def body(x_ref, rW_ref, idx_ref, ew_ref, out_ref,
             ew_all, cnt_all,
             ew_send_sems, ew_recv_sems, cnt_send_sems, cnt_recv_sems):
        p = lax.axis_index("i")

        idx = idx_ref[:, :]
        iota_e = lax.broadcasted_iota(jnp.int32, (T_LOC, LANES), 1)
        oh = (idx == iota_e).astype(jnp.float32)
        counts = jnp.sum(oh, axis=0, keepdims=True)
        cnt_all[pl.ds(p, 1), :] = counts

        barrier_sem = pltpu.get_barrier_semaphore()
        for k in range(1, N_DEV):
            dst = lax.rem(p + k, N_DEV)
            pl.semaphore_signal(
                barrier_sem, inc=1,
                device_id=(dst,), device_id_type=pl.DeviceIdType.MESH,
            )
        pl.semaphore_wait(barrier_sem, N_DEV - 1)

        sends = []
        for k in range(1, N_DEV):
            dst = lax.rem(p + k, N_DEV)
            cnt_rdma = pltpu.make_async_remote_copy(
                src_ref=cnt_all.at[pl.ds(p, 1)],
                dst_ref=cnt_all.at[pl.ds(p, 1)],
                send_sem=cnt_send_sems.at[k - 1],
                recv_sem=cnt_recv_sems.at[k - 1],
                device_id=(dst,), device_id_type=pl.DeviceIdType.MESH,
            )
            cnt_rdma.start()
            sends.append(cnt_rdma)
        for k in range(1, N_DEV):
            dst = lax.rem(p + k, N_DEV)
            ew_rdma = pltpu.make_async_remote_copy(
                src_ref=ew_ref,
                dst_ref=ew_all.at[pl.ds(p * E_LOC, E_LOC)],
                send_sem=ew_send_sems.at[k - 1],
                recv_sem=ew_recv_sems.at[k - 1],
                device_id=(dst,), device_id_type=pl.DeviceIdType.MESH,
            )
            ew_rdma.start()
            sends.append(ew_rdma)

        x_ = x_ref[:, :]
        tri = (
            lax.broadcasted_iota(jnp.int32, (T_LOC, T_LOC), 0)
            > lax.broadcasted_iota(jnp.int32, (T_LOC, T_LOC), 1)
        ).astype(jnp.float32)
        lcb = jnp.dot(tri, oh, preferred_element_type=jnp.float32)

        for k in range(1, N_DEV):
            src_dev = lax.rem(p - k + N_DEV, N_DEV)
            cnt_recv = pltpu.make_async_remote_copy(
                src_ref=cnt_all.at[pl.ds(src_dev, 1)],
                dst_ref=cnt_all.at[pl.ds(src_dev, 1)],
                send_sem=cnt_send_sems.at[k - 1],
                recv_sem=cnt_recv_sems.at[k - 1],
                device_id=(src_dev,), device_id_type=pl.DeviceIdType.MESH,
            )
            cnt_recv.wait_recv()
        dev_before = (
            lax.broadcasted_iota(jnp.int32, (N_DEV, 1), 0) < p
        ).astype(jnp.float32)
        offsets = jnp.sum(cnt_all[:, :] * dev_before, axis=0, keepdims=True)
        before = lcb + offsets
        before_tok = jnp.sum(oh * before, axis=1, keepdims=True)
        keep_row = (before_tok < CAP).astype(jnp.float32)

        acc = jnp.zeros((T_LOC, D_OUT), jnp.float32)
        for j in range(E_LOC):
            m = keep_row * (idx == p * E_LOC + j).astype(jnp.float32)
            acc = acc + jnp.dot(
                x_ * m, ew_ref[j, :, :], preferred_element_type=jnp.float32
            )

        for k in range(1, N_DEV):
            src_dev = lax.rem(p - k + N_DEV, N_DEV)
            ew_recv = pltpu.make_async_remote_copy(
                src_ref=ew_ref,
                dst_ref=ew_all.at[pl.ds(src_dev * E_LOC, E_LOC)],
                send_sem=ew_send_sems.at[k - 1],
                recv_sem=ew_recv_sems.at[k - 1],
                device_id=(src_dev,), device_id_type=pl.DeviceIdType.MESH,
            )
            ew_recv.wait_recv()
            chunk = ew_all[pl.ds(src_dev * E_LOC, E_LOC), :, :]
            for j in range(E_LOC):
                m = keep_row * (idx == src_dev * E_LOC + j).astype(jnp.float32)
                acc = acc + jnp.dot(
                    x_ * m, chunk[j], preferred_element_type=jnp.float32
                )
        out_ref[:, :] = acc

        for rdma in sends:
            rdma.wait_send()

    return pl.pallas_call(
        body,
        out_shape=jax.ShapeDtypeStruct((T_LOC, D_OUT), jnp.float32),
        in_specs=[
            pl.BlockSpec(memory_space=pltpu.VMEM),
            pl.BlockSpec(memory_space=pltpu.VMEM),
            pl.BlockSpec(memory_space=pltpu.VMEM),
            pl.BlockSpec(memory_space=pltpu.VMEM),
        ],
        out_specs=pl.BlockSpec(memory_space=pltpu.VMEM),
        scratch_shapes=[
            pltpu.VMEM((N_EXP, D_IN, D_OUT), jnp.float32),
            pltpu.VMEM((N_DEV, LANES), jnp.float32),
            pltpu.SemaphoreType.DMA((N_DEV - 1,)),
            pltpu.SemaphoreType.DMA((N_DEV - 1,)),
            pltpu.SemaphoreType.DMA((N_DEV - 1,)),
            pltpu.SemaphoreType.DMA((N_DEV - 1,)),
        ],
        compiler_params=pltpu.CompilerParams(collective_id=0),
    )(x, router_W, route_idx, expert_W)


# device time: 15821 ns/iter; 1.5543x vs baseline; 1.5014x over previous
import jax
import jax.numpy as jnp
from jax import lax
from jax.experimental import pallas as pl
from jax.experimental.pallas import tpu as pltpu

N_DEV = 8
N_EXP = 16
E_LOC = 2
T_LOC = 256
D_IN = 128
D_OUT = 256
CAP = 102.0
LANES = 128


def kernel(x, router_W, route_idx, expert_W):
    def body(x_ref, rW_ref, idx_ref, ew_ref, out_ref,
             ew_all, cnt_all,
             ew_send_sems, ew_recv_sems, cnt_send_sems, cnt_recv_sems):
        p = lax.axis_index("i")

        with jax.named_scope("prep"):
            idx = idx_ref[:, :]
            iota_e = lax.broadcasted_iota(jnp.int32, (T_LOC, LANES), 1)
            oh = (idx == iota_e).astype(jnp.float32)
            counts = jnp.sum(oh, axis=0, keepdims=True)
            cnt_all[pl.ds(p, 1), :] = counts
            ew_b = ew_ref[:, :, :].astype(jnp.bfloat16)
            ew_all[pl.ds(p * E_LOC, E_LOC), :, :] = ew_b

        with jax.named_scope("barrier"):
            barrier_sem = pltpu.get_barrier_semaphore()
            for k in range(1, N_DEV):
                dst = lax.rem(p + k, N_DEV)
                pl.semaphore_signal(
                    barrier_sem, inc=1,
                    device_id=(dst,), device_id_type=pl.DeviceIdType.MESH,
                )
            pl.semaphore_wait(barrier_sem, N_DEV - 1)

        sends = []
        with jax.named_scope("issue"):
            for k in range(1, N_DEV):
                dst = lax.rem(p + k, N_DEV)
                cnt_rdma = pltpu.make_async_remote_copy(
                    src_ref=cnt_all.at[pl.ds(p, 1)],
                    dst_ref=cnt_all.at[pl.ds(p, 1)],
                    send_sem=cnt_send_sems.at[k - 1],
                    recv_sem=cnt_recv_sems.at[k - 1],
                    device_id=(dst,), device_id_type=pl.DeviceIdType.MESH,
                )
                cnt_rdma.start()
                sends.append(cnt_rdma)
            for k in range(1, N_DEV):
                dst = lax.rem(p + k, N_DEV)
                ew_rdma = pltpu.make_async_remote_copy(
                    src_ref=ew_all.at[pl.ds(p * E_LOC, E_LOC)],
                    dst_ref=ew_all.at[pl.ds(p * E_LOC, E_LOC)],
                    send_sem=ew_send_sems.at[k - 1],
                    recv_sem=ew_recv_sems.at[k - 1],
                    device_id=(dst,), device_id_type=pl.DeviceIdType.MESH,
                )
                ew_rdma.start()
                sends.append(ew_rdma)

        with jax.named_scope("lcb"):
            xb = x_ref[:, :].astype(jnp.bfloat16)
            tri = (
                lax.broadcasted_iota(jnp.int32, (T_LOC, T_LOC), 0)
                > lax.broadcasted_iota(jnp.int32, (T_LOC, T_LOC), 1)
            ).astype(jnp.float32)
            lcb = jnp.dot(tri, oh, preferred_element_type=jnp.float32)

        with jax.named_scope("cnt_wait"):
            for k in range(1, N_DEV):
                src_dev = lax.rem(p - k + N_DEV, N_DEV)
                cnt_recv = pltpu.make_async_remote_copy(
                    src_ref=cnt_all.at[pl.ds(src_dev, 1)],
                    dst_ref=cnt_all.at[pl.ds(src_dev, 1)],
                    send_sem=cnt_send_sems.at[k - 1],
                    recv_sem=cnt_recv_sems.at[k - 1],
                    device_id=(src_dev,),
                    device_id_type=pl.DeviceIdType.MESH,
                )
                cnt_recv.wait_recv()
        with jax.named_scope("mask"):
            dev_before = (
                lax.broadcasted_iota(jnp.int32, (N_DEV, 1), 0) < p
            ).astype(jnp.float32)
            offsets = jnp.sum(
                cnt_all[:, :] * dev_before, axis=0, keepdims=True)
            before = lcb + offsets
            before_tok = jnp.sum(oh * before, axis=1, keepdims=True)
            keep_row = (before_tok < CAP).astype(jnp.bfloat16)

        with jax.named_scope("local_mm"):
            acc = jnp.zeros((T_LOC, D_OUT), jnp.float32)
            for j in range(E_LOC):
                m = keep_row * (idx == p * E_LOC + j).astype(jnp.bfloat16)
                acc = acc + jnp.dot(
                    xb * m, ew_b[j],
                    preferred_element_type=jnp.float32,
                )

        for k in range(1, N_DEV):
            src_dev = lax.rem(p - k + N_DEV, N_DEV)
            with jax.named_scope(f"ew_wait#k={k}"):
                ew_recv = pltpu.make_async_remote_copy(
                    src_ref=ew_all.at[pl.ds(src_dev * E_LOC, E_LOC)],
                    dst_ref=ew_all.at[pl.ds(src_dev * E_LOC, E_LOC)],
                    send_sem=ew_send_sems.at[k - 1],
                    recv_sem=ew_recv_sems.at[k - 1],
                    device_id=(src_dev,),
                    device_id_type=pl.DeviceIdType.MESH,
                )
                ew_recv.wait_recv()
            with jax.named_scope(f"ew_mm#k={k}"):
                chunk = ew_all[pl.ds(src_dev * E_LOC, E_LOC), :, :]
                for j in range(E_LOC):
                    m = keep_row * (
                        idx == src_dev * E_LOC + j).astype(jnp.bfloat16)
                    acc = acc + jnp.dot(
                        xb * m, chunk[j], preferred_element_type=jnp.float32
                    )
        with jax.named_scope("store_out"):
            out_ref[:, :] = acc

        with jax.named_scope("drain"):
            for rdma in sends:
                rdma.wait_send()

    return pl.pallas_call(
        body,
        out_shape=jax.ShapeDtypeStruct((T_LOC, D_OUT), jnp.float32),
        in_specs=[
            pl.BlockSpec(memory_space=pltpu.VMEM),
            pl.BlockSpec(memory_space=pltpu.VMEM),
            pl.BlockSpec(memory_space=pltpu.VMEM),
            pl.BlockSpec(memory_space=pltpu.VMEM),
        ],
        out_specs=pl.BlockSpec(memory_space=pltpu.VMEM),
        scratch_shapes=[
            pltpu.VMEM((N_EXP, D_IN, D_OUT), jnp.bfloat16),
            pltpu.VMEM((N_DEV, LANES), jnp.float32),
            pltpu.SemaphoreType.DMA((N_DEV - 1,)),
            pltpu.SemaphoreType.DMA((N_DEV - 1,)),
            pltpu.SemaphoreType.DMA((N_DEV - 1,)),
            pltpu.SemaphoreType.DMA((N_DEV - 1,)),
        ],
        compiler_params=pltpu.CompilerParams(collective_id=0),
    )(x, router_W, route_idx, expert_W)
